# baseline (speedup 1.0000x reference)
import functools
import math

import jax
import jax.numpy as jnp
from jax import lax
from jax.experimental import pallas as pl
from jax.experimental.pallas import tpu as pltpu
from jax.experimental.pallas import tpu_sc as plsc

D = 768
N_TOK = 16384
NW = 32
TOK_PER_W = N_TOK // NW
CHUNK = 32
N_CHUNK = TOK_PER_W // CHUNK
LANES = 16
D_SLICES = D // LANES
INV_SQRT = 1.0 / math.sqrt(D)


def _body(widx_h, tidx_h, sidx_h, wt_h, tt_h, st_h, out_h,
          widx_v, tidx_v, sidx_v, rows_v, trows_v, srows_v, gsem):
    wid = lax.axis_index("s") * 2 + lax.axis_index("c")
    base = wid * TOK_PER_W

    pltpu.sync_copy(widx_h.at[pl.ds(base, TOK_PER_W)], widx_v)
    pltpu.sync_copy(tidx_h.at[pl.ds(base, TOK_PER_W)], tidx_v)
    pltpu.sync_copy(sidx_h.at[pl.ds(base, TOK_PER_W)], sidx_v)

    for g in range(N_CHUNK):
        sl_tok = pl.ds(g * CHUNK, CHUNK)
        cw = pltpu.async_copy(wt_h.at[widx_v.at[sl_tok]], rows_v, gsem)
        ct = pltpu.async_copy(tt_h.at[tidx_v.at[sl_tok]], trows_v, gsem)
        cs = pltpu.async_copy(st_h.at[sidx_v.at[sl_tok]], srows_v, gsem)
        cw.wait()
        ct.wait()
        cs.wait()

        def add_one(i, carry):
            t = i // D_SLICES
            sl = pl.ds((i % D_SLICES) * LANES, LANES)
            rows_v[t, sl] = rows_v[t, sl] + (trows_v[t, sl] + srows_v[t, sl]) * INV_SQRT
            return carry
        lax.fori_loop(0, CHUNK * D_SLICES, add_one, 0)

        pltpu.sync_copy(rows_v, out_h.at[pl.ds(base + g * CHUNK, CHUNK)])


def kernel(word_input, position_input, task_input, segment_input,
           word_table, task_table, segment_table):
    del position_input
    w = word_input.reshape(-1).astype(jnp.int32)
    t = task_input.reshape(-1).astype(jnp.int32)
    s = segment_input.reshape(-1).astype(jnp.int32)
    mesh = plsc.VectorSubcoreMesh(core_axis_name="c", subcore_axis_name="s")
    f = functools.partial(
        pl.kernel,
        mesh=mesh,
        out_type=jax.ShapeDtypeStruct((N_TOK, D), jnp.float32),
        scratch_types=[
            pltpu.VMEM((TOK_PER_W,), jnp.int32),
            pltpu.VMEM((TOK_PER_W,), jnp.int32),
            pltpu.VMEM((TOK_PER_W,), jnp.int32),
            pltpu.VMEM((CHUNK, D), jnp.float32),
            pltpu.VMEM((CHUNK, D), jnp.float32),
            pltpu.VMEM((CHUNK, D), jnp.float32),
            pltpu.SemaphoreType.DMA,
        ],
    )(_body)
    out = f(w, t, s, word_table, task_table, segment_table)
    return out.reshape(word_input.shape[0], word_input.shape[1], D)


# speedup vs baseline: 3.6695x; 3.6695x over previous
import functools
import math

import jax
import jax.numpy as jnp
from jax import lax
from jax.experimental import pallas as pl
from jax.experimental.pallas import tpu as pltpu
from jax.experimental.pallas import tpu_sc as plsc

D = 768
N_TOK = 16384
NW = 32
TOK_PER_W = N_TOK // NW
CHUNK = 64
N_CHUNK = TOK_PER_W // CHUNK
LANES = 16
D_SLICES = D // LANES
INV_SQRT = 1.0 / math.sqrt(D)


def _body(widx_h, tidx_h, sidx_h, wt_h, tt_h, st_h, out_h,
          tt_v, st_v, combo_v, widx_v, c_v, tmp_v, rows0, rows1, gsem, osem):
    wid = lax.axis_index("s") * 2 + lax.axis_index("c")
    base = wid * TOK_PER_W
    rows = (rows0, rows1)

    pltpu.sync_copy(tt_h, tt_v)
    pltpu.sync_copy(st_h, st_v)
    for t in range(3):
        for s in range(3):
            def build(d, carry, t=t, s=s):
                sl = pl.ds(d * LANES, LANES)
                combo_v[t * 3 + s, sl] = (tt_v[t, sl] + st_v[s, sl]) * INV_SQRT
                return carry
            lax.fori_loop(0, D_SLICES, build, 0)

    pltpu.sync_copy(widx_h.at[pl.ds(base, TOK_PER_W)], widx_v)
    pltpu.sync_copy(tidx_h.at[pl.ds(base, TOK_PER_W)], c_v.at[pl.ds(0, TOK_PER_W)])
    pltpu.sync_copy(sidx_h.at[pl.ds(base, TOK_PER_W)], tmp_v)

    def cidx(i, carry):
        sl = pl.ds(i * LANES, LANES)
        c_v[sl] = c_v[sl] * 3 + tmp_v[sl]
        return carry
    lax.fori_loop(0, TOK_PER_W // LANES, cidx, 0)

    def start_gather(g):
        idx = widx_v.at[pl.ds(g * CHUNK, CHUNK)]
        return pltpu.async_copy(wt_h.at[idx], rows[g % 2], gsem)

    gcopies = [None, None]
    ocopies = [None, None]
    gcopies[0] = start_gather(0)
    for g in range(N_CHUNK):
        b = g % 2
        if g + 1 < N_CHUNK:
            nb = (g + 1) % 2
            if ocopies[nb] is not None:
                ocopies[nb].wait()
            gcopies[nb] = start_gather(g + 1)
        gcopies[b].wait()

        def add_tok(tok, carry, b=b, g=g):
            c = c_v[pl.ds(g * CHUNK + tok, LANES)][0]
            for k in range(D_SLICES):
                sl = pl.ds(k * LANES, LANES)
                plsc.addupdate(rows[b].at[tok, sl], combo_v[c, sl])
            return carry
        lax.fori_loop(0, CHUNK, add_tok, 0)

        ocopies[b] = pltpu.async_copy(
            rows[b], out_h.at[pl.ds(base + g * CHUNK, CHUNK)], osem)
    for oc in ocopies:
        if oc is not None:
            oc.wait()


def kernel(word_input, position_input, task_input, segment_input,
           word_table, task_table, segment_table):
    del position_input
    w = word_input.reshape(-1).astype(jnp.int32)
    t = task_input.reshape(-1).astype(jnp.int32)
    s = segment_input.reshape(-1).astype(jnp.int32)
    mesh = plsc.VectorSubcoreMesh(core_axis_name="c", subcore_axis_name="s")
    f = functools.partial(
        pl.kernel,
        mesh=mesh,
        out_type=jax.ShapeDtypeStruct((N_TOK, D), jnp.float32),
        scratch_types=[
            pltpu.VMEM((3, D), jnp.float32),
            pltpu.VMEM((3, D), jnp.float32),
            pltpu.VMEM((9, D), jnp.float32),
            pltpu.VMEM((TOK_PER_W,), jnp.int32),
            pltpu.VMEM((TOK_PER_W + LANES,), jnp.int32),
            pltpu.VMEM((TOK_PER_W,), jnp.int32),
            pltpu.VMEM((CHUNK, D), jnp.float32),
            pltpu.VMEM((CHUNK, D), jnp.float32),
            pltpu.SemaphoreType.DMA,
            pltpu.SemaphoreType.DMA,
        ],
    )(_body)
    out = f(w, t, s, word_table, task_table, segment_table)
    return out.reshape(word_input.shape[0], word_input.shape[1], D)


# speedup vs baseline: 5.7590x; 1.5694x over previous
import functools
import math

import jax
import jax.numpy as jnp
from jax import lax
from jax.experimental import pallas as pl
from jax.experimental.pallas import tpu as pltpu
from jax.experimental.pallas import tpu_sc as plsc

D = 768
N_TOK = 16384
NW = 32
TOK_PER_W = N_TOK // NW
CHUNK = 64
N_CHUNK = TOK_PER_W // CHUNK
NBUF = 2
LANES = 16
D_SLICES = D // LANES
GW = 6
INV_SQRT = 1.0 / math.sqrt(D)


def _body(widx_h, tidx_h, sidx_h, wt_h, tt_h, st_h, out_h,
          tt_v, st_v, combo_v, widx_v, c_v, tmp_v, rows0, rows1,
          gsem, osem):
    wid = lax.axis_index("s") * 2 + lax.axis_index("c")
    base = wid * TOK_PER_W
    rows = (rows0, rows1)

    pltpu.sync_copy(tt_h, tt_v)
    pltpu.sync_copy(st_h, st_v)
    for t in range(3):
        for s in range(3):
            def build(d, carry, t=t, s=s):
                sl = pl.ds(d * LANES, LANES)
                combo_v[t * 3 + s, sl] = (tt_v[t, sl] + st_v[s, sl]) * INV_SQRT
                return carry
            lax.fori_loop(0, D_SLICES, build, 0)

    pltpu.sync_copy(widx_h.at[pl.ds(base, TOK_PER_W)], widx_v)
    pltpu.sync_copy(tidx_h.at[pl.ds(base, TOK_PER_W)], c_v.at[pl.ds(0, TOK_PER_W)])
    pltpu.sync_copy(sidx_h.at[pl.ds(base, TOK_PER_W)], tmp_v)

    def cidx(i, carry):
        sl = pl.ds(i * LANES, LANES)
        c_v[sl] = c_v[sl] * 3 + tmp_v[sl]
        return carry
    lax.fori_loop(0, TOK_PER_W // LANES, cidx, 0)

    def start_gather(g):
        idx = widx_v.at[pl.ds(g * CHUNK, CHUNK)]
        return pltpu.async_copy(wt_h.at[idx], rows[g % NBUF], gsem)

    def add_chunk(g, rows_b):
        @plsc.parallel_loop(0, CHUNK, unroll=2)
        def tok_body(tok, g=g, rows_b=rows_b):
            c = c_v[pl.ds(g * CHUNK + tok, LANES)][0]
            for k in range(D_SLICES):
                sl = pl.ds(k * LANES, LANES)
                plsc.addupdate(rows_b.at[tok, sl], combo_v[c, sl])

    gcopies = [None] * NBUF
    ocopies = [None] * NBUF
    gcopies[0] = start_gather(0)
    for g in range(N_CHUNK):
        b = g % NBUF
        if g + 1 < N_CHUNK:
            nb = (g + 1) % NBUF
            if ocopies[nb] is not None:
                ocopies[nb].wait()
            gcopies[nb] = start_gather(g + 1)
        gcopies[b].wait()
        add_chunk(g, rows[b])
        ocopies[b] = pltpu.async_copy(
            rows[b], out_h.at[pl.ds(base + g * CHUNK, CHUNK)], osem)
    for oc in ocopies:
        if oc is not None:
            oc.wait()


def kernel(word_input, position_input, task_input, segment_input,
           word_table, task_table, segment_table):
    del position_input
    w = word_input.reshape(-1).astype(jnp.int32)
    t = task_input.reshape(-1).astype(jnp.int32)
    s = segment_input.reshape(-1).astype(jnp.int32)
    mesh = plsc.VectorSubcoreMesh(core_axis_name="c", subcore_axis_name="s")
    f = functools.partial(
        pl.kernel,
        mesh=mesh,
        out_type=jax.ShapeDtypeStruct((N_TOK, D), jnp.float32),
        scratch_types=[
            pltpu.VMEM((3, D), jnp.float32),
            pltpu.VMEM((3, D), jnp.float32),
            pltpu.VMEM((9, D), jnp.float32),
            pltpu.VMEM((TOK_PER_W,), jnp.int32),
            pltpu.VMEM((TOK_PER_W + 2 * LANES,), jnp.int32),
            pltpu.VMEM((TOK_PER_W,), jnp.int32),
            pltpu.VMEM((CHUNK, D), jnp.float32),
            pltpu.VMEM((CHUNK, D), jnp.float32),
            pltpu.SemaphoreType.DMA,
            pltpu.SemaphoreType.DMA,
        ],
    )(_body)
    out = f(w, t, s, word_table, task_table, segment_table)
    return out.reshape(word_input.shape[0], word_input.shape[1], D)


# speedup vs baseline: 6.7888x; 1.1788x over previous
import functools
import math

import jax
import jax.numpy as jnp
from jax import lax
from jax.experimental import pallas as pl
from jax.experimental.pallas import tpu as pltpu
from jax.experimental.pallas import tpu_sc as plsc

D = 768
N_TOK = 16384
NW = 32
TOK_PER_W = N_TOK // NW
CHUNK = 64
N_CHUNK = TOK_PER_W // CHUNK
NBUF = 2
LANES = 16
D_SLICES = D // LANES
GW = 6
INV_SQRT = 1.0 / math.sqrt(D)


def _body(widx_h, tidx_h, sidx_h, wt_h, tt_h, st_h, out_h,
          tt_v, st_v, combo_v, widx_v, c_v, tmp_v, rows0, rows1,
          gsem, osem):
    wid = lax.axis_index("s") * 2 + lax.axis_index("c")
    base = wid * TOK_PER_W
    rows = (rows0, rows1)

    pltpu.sync_copy(tt_h, tt_v)
    pltpu.sync_copy(st_h, st_v)
    for t in range(3):
        for s in range(3):
            def build(d, carry, t=t, s=s):
                sl = pl.ds(d * LANES, LANES)
                combo_v[t * 3 + s, sl] = (tt_v[t, sl] + st_v[s, sl]) * INV_SQRT
                return carry
            lax.fori_loop(0, D_SLICES, build, 0)

    pltpu.sync_copy(widx_h.at[pl.ds(base, TOK_PER_W)], widx_v)
    pltpu.sync_copy(tidx_h.at[pl.ds(base, TOK_PER_W)], c_v.at[pl.ds(0, TOK_PER_W)])
    pltpu.sync_copy(sidx_h.at[pl.ds(base, TOK_PER_W)], tmp_v)

    def cidx(i, carry):
        sl = pl.ds(i * LANES, LANES)
        c_v[sl] = c_v[sl] * 3 + tmp_v[sl]
        return carry
    lax.fori_loop(0, TOK_PER_W // LANES, cidx, 0)

    def start_gather(g):
        idx = widx_v.at[pl.ds(g * CHUNK, CHUNK)]
        return pltpu.async_copy(wt_h.at[idx], rows[g % NBUF], gsem)

    def add_chunk(g, rows_b):
        def tok_body(tok, c_cur, g=g, rows_b=rows_b):
            c_next = c_v[pl.ds(g * CHUNK + tok + 1, LANES)][0]
            vals = [combo_v[c_cur, pl.ds(k * LANES, LANES)] for k in range(GW)]
            for k in range(D_SLICES):
                if k + GW < D_SLICES:
                    vals.append(combo_v[c_cur, pl.ds((k + GW) * LANES, LANES)])
                plsc.addupdate(rows_b.at[tok, pl.ds(k * LANES, LANES)],
                               vals[k])
            return c_next
        c0 = c_v[pl.ds(g * CHUNK, LANES)][0]
        lax.fori_loop(0, CHUNK, tok_body, c0)

    gcopies = [None] * NBUF
    ocopies = [None] * NBUF
    gcopies[0] = start_gather(0)
    for g in range(N_CHUNK):
        b = g % NBUF
        if g + 1 < N_CHUNK:
            nb = (g + 1) % NBUF
            if ocopies[nb] is not None:
                ocopies[nb].wait()
            gcopies[nb] = start_gather(g + 1)
        gcopies[b].wait()
        add_chunk(g, rows[b])
        ocopies[b] = pltpu.async_copy(
            rows[b], out_h.at[pl.ds(base + g * CHUNK, CHUNK)], osem)
    for oc in ocopies:
        if oc is not None:
            oc.wait()


def kernel(word_input, position_input, task_input, segment_input,
           word_table, task_table, segment_table):
    del position_input
    w = word_input.reshape(-1).astype(jnp.int32)
    t = task_input.reshape(-1).astype(jnp.int32)
    s = segment_input.reshape(-1).astype(jnp.int32)
    mesh = plsc.VectorSubcoreMesh(core_axis_name="c", subcore_axis_name="s")
    f = functools.partial(
        pl.kernel,
        mesh=mesh,
        out_type=jax.ShapeDtypeStruct((N_TOK, D), jnp.float32),
        scratch_types=[
            pltpu.VMEM((3, D), jnp.float32),
            pltpu.VMEM((3, D), jnp.float32),
            pltpu.VMEM((9, D), jnp.float32),
            pltpu.VMEM((TOK_PER_W,), jnp.int32),
            pltpu.VMEM((TOK_PER_W + 2 * LANES,), jnp.int32),
            pltpu.VMEM((TOK_PER_W,), jnp.int32),
            pltpu.VMEM((CHUNK, D), jnp.float32),
            pltpu.VMEM((CHUNK, D), jnp.float32),
            pltpu.SemaphoreType.DMA,
            pltpu.SemaphoreType.DMA,
        ],
    )(_body)
    out = f(w, t, s, word_table, task_table, segment_table)
    return out.reshape(word_input.shape[0], word_input.shape[1], D)
